# baseline (speedup 1.0000x reference)
import functools

import jax
import jax.numpy as jnp
from jax import lax
from jax.experimental import pallas as pl
from jax.experimental.pallas import tpu as pltpu
from jax.experimental.pallas import tpu_sc as plsc

_B = 16384
_T = 129
_L = 16
_NC = 1
_NS = 16
_NW = _NC * _NS
_BPW = _B // _NW


def _build():
    mesh = plsc.VectorSubcoreMesh(
        core_axis_name="c", subcore_axis_name="s",
        num_cores=_NC, num_subcores=_NS,
    )

    @functools.partial(
        pl.kernel,
        mesh=mesh,
        out_type=jax.ShapeDtypeStruct((_B,), jnp.float32),
        scratch_types=[
            pltpu.VMEM((_T,), jnp.float32),
            pltpu.VMEM((_BPW,), jnp.int32),
            pltpu.VMEM((_BPW,), jnp.float32),
            pltpu.SemaphoreType.DMA,
            pltpu.SemaphoreType.DMA,
            pltpu.SemaphoreType.DMA,
        ],
        compiler_params=pltpu.CompilerParams(needs_layout_passes=False),
    )
    def gather_kernel(start_hbm, table_hbm, out_hbm,
                      table_v, idx_v, out_v, sem_t, sem_i, sem_o):
        wid = lax.axis_index("s") * _NC + lax.axis_index("c")
        base = wid * _BPW
        ct = pltpu.async_copy(table_hbm, table_v, sem_t)
        ci = pltpu.async_copy(start_hbm.at[pl.ds(base, _BPW)], idx_v, sem_i)
        ci.wait()
        ct.wait()
        half = _BPW // 2
        pending = []
        for h in range(2):
            @plsc.parallel_loop(0, half, step=_L, unroll=16)
            def body(off, h=h):
                idx = idx_v[pl.ds(h * half + off, _L)]
                out_v[pl.ds(h * half + off, _L)] = plsc.load_gather(table_v, [idx])
            pending.append(pltpu.async_copy(
                out_v.at[pl.ds(h * half, half)],
                out_hbm.at[pl.ds(base + h * half, half)], sem_o))
        for c in pending:
            c.wait()

    return gather_kernel


_gather = _build()


def kernel(start, pos_bias):
    return _gather(start.astype(jnp.int32), pos_bias)


# speedup vs baseline: 1.0071x; 1.0071x over previous
import functools

import jax
import jax.numpy as jnp
from jax import lax
from jax.experimental import pallas as pl
from jax.experimental.pallas import tpu as pltpu
from jax.experimental.pallas import tpu_sc as plsc

_B = 16384
_T = 129
_L = 16
_NC = 1
_NS = 16
_NW = _NC * _NS
_BPW = _B // _NW


def _build():
    mesh = plsc.VectorSubcoreMesh(
        core_axis_name="c", subcore_axis_name="s",
        num_cores=_NC, num_subcores=_NS,
    )

    @functools.partial(
        pl.kernel,
        mesh=mesh,
        out_type=jax.ShapeDtypeStruct((_B,), jnp.float32),
        scratch_types=[
            pltpu.VMEM((_T,), jnp.float32),
            pltpu.VMEM((_BPW,), jnp.int32),
            pltpu.VMEM((_BPW,), jnp.float32),
            pltpu.SemaphoreType.DMA,
            pltpu.SemaphoreType.DMA,
            pltpu.SemaphoreType.DMA,
        ],
        compiler_params=pltpu.CompilerParams(needs_layout_passes=False),
    )
    def gather_kernel(start_hbm, table_hbm, out_hbm,
                      table_v, idx_v, out_v, sem_t, sem_i, sem_o):
        wid = lax.axis_index("s") * _NC + lax.axis_index("c")
        base = wid * _BPW
        ct = pltpu.async_copy(table_hbm, table_v, sem_t)
        ci = pltpu.async_copy(start_hbm.at[pl.ds(base, _BPW)], idx_v, sem_i)
        ci.wait()
        ct.wait()
        chunk = _BPW // 4
        pending = []
        for h in range(4):
            @plsc.parallel_loop(0, chunk, step=_L, unroll=8)
            def body(off, h=h):
                idx = idx_v[pl.ds(h * chunk + off, _L)]
                out_v[pl.ds(h * chunk + off, _L)] = plsc.load_gather(table_v, [idx])
            pending.append(pltpu.async_copy(
                out_v.at[pl.ds(h * chunk, chunk)],
                out_hbm.at[pl.ds(base + h * chunk, chunk)], sem_o))
        for c in pending:
            c.wait()

    return gather_kernel


_gather = _build()


def kernel(start, pos_bias):
    return _gather(start.astype(jnp.int32), pos_bias)
